# baseline (speedup 1.0000x reference)
import functools

import jax
import jax.numpy as jnp
from jax import lax
from jax.experimental import pallas as pl
from jax.experimental.pallas import tpu as pltpu
from jax.experimental.pallas import tpu_sc as plsc

NB = 256
N_PTS = 2097152
MROWS = N_PTS // 128


_R1 = 256
_GRID1 = MROWS // _R1


def _idx_body(x_ref, y_ref, z_ref, idx_ref):
    x = x_ref[...]
    y = y_ref[...]
    z = z_ref[...]
    cond = ((jnp.abs(x) < 0.5) & (jnp.abs(y) < 0.5) & (jnp.abs(z) < 0.5))
    ix = jnp.clip((x * 256.0 + 128.0).astype(jnp.int32), 0, NB - 1)
    iy = jnp.clip((y * 256.0 + 128.0).astype(jnp.int32), 0, NB - 1)
    iz = jnp.clip((z * 256.0 + 128.0).astype(jnp.int32), 0, NB - 1)
    base = (ix * NB + iy) * 1024 + (iz >> 7) * 512 + (iz & 127)
    idx_ref[...] = jnp.where(cond, base, -1)


_idx_kernel = pl.pallas_call(
    _idx_body,
    grid=(_GRID1,),
    in_specs=[pl.BlockSpec((_R1, 128), lambda i: (i, 0))] * 3,
    out_specs=pl.BlockSpec((_R1, 128), lambda i: (i, 0)),
    out_shape=jax.ShapeDtypeStruct((MROWS, 128), jnp.int32),
)


NC, NS, L = 2, 16, 16
NW = NC * NS
PER_W = N_PTS // NW
CHUNK = 2048
N_CHUNKS = PER_W // CHUNK
N_PAIRS = N_CHUNKS // 2
GROUPS = CHUNK // L
GSEG = 256
MAXSEG = CHUNK // GSEG

_mesh = plsc.VectorSubcoreMesh(
    core_axis_name="c", subcore_axis_name="s", num_cores=NC, num_subcores=NS
)


@functools.partial(
    pl.kernel,
    out_type=jax.ShapeDtypeStruct((4, N_PTS), jnp.float32),
    mesh=_mesh,
    scratch_types=[
        pltpu.VMEM((2, CHUNK), jnp.int32),
        pltpu.VMEM((2, CHUNK), jnp.int32),
        pltpu.VMEM((2, CHUNK), jnp.int32),
        pltpu.VMEM((2, CHUNK), jnp.int32),
        pltpu.VMEM((2, CHUNK), jnp.int32),
        pltpu.VMEM((2, CHUNK), jnp.int32),
        pltpu.VMEM((2, CHUNK), jnp.float32),
        pltpu.VMEM((2, CHUNK), jnp.float32),
        pltpu.VMEM((2, CHUNK), jnp.float32),
        pltpu.VMEM((2, CHUNK), jnp.float32),
        pltpu.VMEM((2, CHUNK), jnp.float32),
        pltpu.VMEM((2, CHUNK), jnp.float32),
        pltpu.VMEM((2, CHUNK), jnp.float32),
        pltpu.VMEM((2, CHUNK), jnp.float32),
        pltpu.SemaphoreType.DMA,
        pltpu.SemaphoreType.DMA,
        pltpu.SemaphoreType.DMA,
        pltpu.SemaphoreType.DMA,
        pltpu.SemaphoreType.DMA,
        pltpu.SemaphoreType.DMA,
    ],
    compiler_params=pltpu.CompilerParams(use_tc_tiling_on_sc=False,
                                         needs_layout_passes=False),
)
def _gather_sc(idx_hbm, vox_hbm, out_hbm,
               ib_v, rk_v, b0_v, b1_v, b2_v, b3_v,
               v0_v, v1_v, v2_v, v3_v, e0_v, e1_v, e2_v, e3_v,
               isem0, isem1, gsem0, gsem1, osem0, osem1):
    wid = lax.axis_index("s") * NC + lax.axis_index("c")
    isem = (isem0, isem1)
    gsem = (gsem0, gsem1)
    osem = (osem0, osem1)
    cbufs = (b0_v, b1_v, b2_v, b3_v)
    vbufs = (v0_v, v1_v, v2_v, v3_v)
    ebufs = (e0_v, e1_v, e2_v, e3_v)

    def cbase(ci):
        return wid * PER_W + ci * CHUNK

    def in_copy(par, ci):
        return pltpu.make_async_copy(
            idx_hbm.at[pl.ds(cbase(ci), CHUNK)], ib_v.at[par], isem[par])

    def seg_copies(par, j):
        sl = pl.ds(j * GSEG, GSEG)
        return [pltpu.make_async_copy(
                    vox_hbm.at[cbufs[c].at[par].at[sl]],
                    vbufs[c].at[par].at[sl], gsem[par])
                for c in range(4)]

    def out_copies(par, ci):
        return [pltpu.make_async_copy(
                    ebufs[c].at[par],
                    out_hbm.at[c, pl.ds(cbase(ci), CHUNK)], osem[par])
                for c in range(4)]

    def compact(par):
        def body(g, cnt):
            sl = pl.ds(g * L, L)
            w = ib_v[par, sl]
            m = w >= 0
            pre = plsc.cumsum(m.astype(jnp.int32))
            rank = cnt + pre - 1
            rk_v[par, sl] = jnp.maximum(rank, 0)
            plsc.store_scatter(b0_v.at[par], [rank], w, mask=m)
            plsc.store_scatter(b1_v.at[par], [rank], w + 128, mask=m)
            plsc.store_scatter(b2_v.at[par], [rank], w + 256, mask=m)
            plsc.store_scatter(b3_v.at[par], [rank], w + 384, mask=m)
            return cnt + plsc.all_reduce_population_count(m)
        cnt = lax.fori_loop(0, GROUPS, body,
                            jnp.zeros((L,), jnp.int32), unroll=4)
        return (jnp.max(cnt) + GSEG - 1) >> 8

    def fire_gathers(par, nseg):
        def body(j, carry):
            for cp in seg_copies(par, j):
                cp.start()
            return carry
        lax.fori_loop(0, nseg, body, 0)

    def wait_gathers(par, nseg):
        def body(j, carry):
            for cp in seg_copies(par, j):
                cp.wait()
            return carry
        lax.fori_loop(0, nseg, body, 0)

    def expand(par):
        def body(g, carry):
            sl = pl.ds(g * L, L)
            rk = rk_v[par, sl]
            for c in range(4):
                ebufs[c][par, sl] = plsc.load_gather(
                    vbufs[c].at[par], [rk])
            return carry
        lax.fori_loop(0, GROUPS, body, 0, unroll=4)

    def prefill(g, carry):
        sl = pl.ds(g * L, L)
        zero = jnp.zeros((L,), jnp.int32)
        for par in (0, 1):
            for c in range(4):
                cbufs[c][par, sl] = zero
        return carry
    lax.fori_loop(0, GROUPS, prefill, 0)

    in_copy(0, 0).start()

    def pair_body(p, carry):
        cnts = list(carry)
        for par in (0, 1):
            ci = 2 * p + par
            oth = 1 - par
            in_copy(par, ci).wait()
            cnts[par] = compact(par)
            fire_gathers(par, cnts[par])

            @pl.when(ci >= 1)
            def _():
                wait_gathers(oth, cnts[oth])

            @pl.when(ci >= 3)
            def _():
                for cp in out_copies(oth, ci - 3):
                    cp.wait()

            @pl.when(ci >= 1)
            def _():
                expand(oth)
                for cp in out_copies(oth, ci - 1):
                    cp.start()

            @pl.when(ci + 1 <= N_CHUNKS - 1)
            def _():
                in_copy(oth, ci + 1).start()
        return tuple(cnts)

    cnts = lax.fori_loop(
        0, N_PAIRS, pair_body,
        (jnp.zeros((), jnp.int32), jnp.zeros((), jnp.int32)))

    last = N_CHUNKS - 1
    wait_gathers(1, cnts[1])
    for cp in out_copies(1, last - 2):
        cp.wait()
    expand(1)
    for cp in out_copies(1, last):
        cp.start()
    for cp in out_copies(0, last - 1):
        cp.wait()
    for cp in out_copies(1, last):
        cp.wait()


_R2 = 256
_GRID2 = MROWS // _R2


def _act_body(idx_ref, r_ref, g_ref, b_ref, s_ref,
              ro_ref, go_ref, bo_ref, so_ref):
    cond = idx_ref[...] >= 0
    zero = jnp.float32(0.0)
    r = jnp.where(cond, r_ref[...], zero)
    g = jnp.where(cond, g_ref[...], zero)
    b = jnp.where(cond, b_ref[...], zero)
    s = jnp.where(cond, s_ref[...], zero)
    ro_ref[...] = jax.nn.sigmoid(r)
    go_ref[...] = jax.nn.sigmoid(g)
    bo_ref[...] = jax.nn.sigmoid(b)
    so_ref[...] = jnp.maximum(s, zero)


_act_kernel = pl.pallas_call(
    _act_body,
    grid=(_GRID2,),
    in_specs=[
        pl.BlockSpec((_R2, 128), lambda i: (i, 0)),
        pl.BlockSpec((_R2, 128), lambda i: (i, 0)),
        pl.BlockSpec((_R2, 128), lambda i: (i + _GRID2, 0)),
        pl.BlockSpec((_R2, 128), lambda i: (i + 2 * _GRID2, 0)),
        pl.BlockSpec((_R2, 128), lambda i: (i + 3 * _GRID2, 0)),
    ],
    out_specs=[pl.BlockSpec((_R2, 128), lambda i: (i, 0))] * 4,
    out_shape=[jax.ShapeDtypeStruct((MROWS, 128), jnp.float32)] * 4,
)


def kernel(xyz, voxels):
    x = xyz[:, 0].reshape(MROWS, 128)
    y = xyz[:, 1].reshape(MROWS, 128)
    z = xyz[:, 2].reshape(MROWS, 128)
    vox_lin = (voxels.reshape(NB, NB, 2, 128, 4)
               .transpose(0, 1, 2, 4, 3)
               .reshape(NB * NB * NB * 4))
    idx = _idx_kernel(x, y, z)
    g4 = _gather_sc(idx.reshape(N_PTS), vox_lin)
    g4v = g4.reshape(4 * MROWS, 128)
    rp, gp, bp, sp = _act_kernel(idx, g4v, g4v, g4v, g4v)
    rgb = jnp.stack(
        [rp.reshape(N_PTS), gp.reshape(N_PTS), bp.reshape(N_PTS)], axis=1)
    return rgb, sp.reshape(N_PTS, 1)


# speedup vs baseline: 6.3575x; 6.3575x over previous
import functools

import jax
import jax.numpy as jnp
from jax import lax
from jax.experimental import pallas as pl
from jax.experimental.pallas import tpu as pltpu
from jax.experimental.pallas import tpu_sc as plsc

NB = 256
N_PTS = 2097152
MROWS = N_PTS // 128


_R1 = 256
_GRID1 = MROWS // _R1


def _idx_body(x_ref, y_ref, z_ref, idx_ref):
    x = x_ref[...]
    y = y_ref[...]
    z = z_ref[...]
    cond = ((jnp.abs(x) < 0.5) & (jnp.abs(y) < 0.5) & (jnp.abs(z) < 0.5))
    ix = jnp.clip((x * 256.0 + 128.0).astype(jnp.int32), 0, NB - 1)
    iy = jnp.clip((y * 256.0 + 128.0).astype(jnp.int32), 0, NB - 1)
    iz = jnp.clip((z * 256.0 + 128.0).astype(jnp.int32), 0, NB - 1)
    base = (ix * NB + iy) * 1024 + (iz >> 7) * 512 + (iz & 127)
    idx_ref[...] = jnp.where(cond, base, -1)


_idx_kernel = pl.pallas_call(
    _idx_body,
    grid=(_GRID1,),
    in_specs=[pl.BlockSpec((_R1, 128), lambda i: (i, 0))] * 3,
    out_specs=pl.BlockSpec((_R1, 128), lambda i: (i, 0)),
    out_shape=jax.ShapeDtypeStruct((MROWS, 128), jnp.int32),
)


NC, NS, L = 2, 16, 16
NW = NC * NS
PER_W = N_PTS // NW
CHUNK = 2048
N_CHUNKS = PER_W // CHUNK
N_PAIRS = N_CHUNKS // 2
GROUPS = CHUNK // L
GATHER_SEG = 512
N_SEG = CHUNK // GATHER_SEG

_mesh = plsc.VectorSubcoreMesh(
    core_axis_name="c", subcore_axis_name="s", num_cores=NC, num_subcores=NS
)


@functools.partial(
    pl.kernel,
    out_type=jax.ShapeDtypeStruct((4, N_PTS), jnp.float32),
    mesh=_mesh,
    scratch_types=[
        pltpu.VMEM((2, CHUNK), jnp.int32),
        pltpu.VMEM((2, CHUNK), jnp.int32),
        pltpu.VMEM((2, CHUNK), jnp.int32),
        pltpu.VMEM((2, CHUNK), jnp.int32),
        pltpu.VMEM((2, CHUNK), jnp.float32),
        pltpu.VMEM((2, CHUNK), jnp.float32),
        pltpu.VMEM((2, CHUNK), jnp.float32),
        pltpu.VMEM((2, CHUNK), jnp.float32),
        pltpu.SemaphoreType.DMA,
        pltpu.SemaphoreType.DMA,
        pltpu.SemaphoreType.DMA,
        pltpu.SemaphoreType.DMA,
        pltpu.SemaphoreType.DMA,
        pltpu.SemaphoreType.DMA,
    ],
    compiler_params=pltpu.CompilerParams(use_tc_tiling_on_sc=False),
)
def _gather_sc(idx_hbm, vox_hbm, out_hbm,
               ib_v, o1_v, o2_v, o3_v, c0_v, c1_v, c2_v, c3_v,
               isem0, isem1, gsem0, gsem1, osem0, osem1):
    wid = lax.axis_index("s") * NC + lax.axis_index("c")
    isem = (isem0, isem1)
    gsem = (gsem0, gsem1)
    osem = (osem0, osem1)

    def cbase(ci):
        return wid * PER_W + ci * CHUNK

    def gather_copies(par, ci):
        out = []
        for c, ob in enumerate([ib_v, o1_v, o2_v, o3_v]):
            dst = (c0_v, c1_v, c2_v, c3_v)[c]
            for j in range(N_SEG):
                sl = pl.ds(j * GATHER_SEG, GATHER_SEG)
                out.append(pltpu.make_async_copy(
                    vox_hbm.at[plsc.Indices(ob.at[par].at[sl],
                                            ignored_value=128 * c - 1)],
                    dst.at[par].at[sl], gsem[par]))
        return out

    def out_copies(par, ci):
        return [pltpu.make_async_copy(
                    (c0_v, c1_v, c2_v, c3_v)[c].at[par],
                    out_hbm.at[c, pl.ds(cbase(ci), CHUNK)], osem[par])
                for c in range(4)]

    def in_copy(par, ci):
        return pltpu.make_async_copy(
            idx_hbm.at[pl.ds(cbase(ci), CHUNK)], ib_v.at[par], isem[par])

    in_copy(0, 0).start()

    def pair_body(p, carry):
        for par in (0, 1):
            ci = 2 * p + par
            oth = 1 - par
            in_copy(par, ci).wait()
            def off_body(g, carry2, ob=None, c=0):
                w = ib_v[par, pl.ds(g * L, L)]
                ob[par, pl.ds(g * L, L)] = w + 128 * c
                return carry2
            for c, ob in enumerate([o1_v, o2_v, o3_v], start=1):
                lax.fori_loop(0, GROUPS,
                              functools.partial(off_body, ob=ob, c=c), 0,
                              unroll=4)
            @pl.when(ci >= 2)
            def _():
                for cp in out_copies(par, ci - 2):
                    cp.wait()
            for cp in gather_copies(par, ci):
                cp.start()
            @pl.when(ci >= 1)
            def _():
                for cp in gather_copies(oth, ci - 1):
                    cp.wait()
                for cp in out_copies(oth, ci - 1):
                    cp.start()
            @pl.when(ci + 1 <= N_CHUNKS - 1)
            def _():
                in_copy(oth, ci + 1).start()
        return carry

    lax.fori_loop(0, N_PAIRS, pair_body, 0)

    last = N_CHUNKS - 1
    for cp in gather_copies(1, last):
        cp.wait()
    for cp in out_copies(1, last):
        cp.start()
    for cp in out_copies(0, last - 1):
        cp.wait()
    for cp in out_copies(1, last):
        cp.wait()


_R2 = 256
_GRID2 = MROWS // _R2


def _act_body(idx_ref, r_ref, g_ref, b_ref, s_ref,
              ro_ref, go_ref, bo_ref, so_ref):
    cond = idx_ref[...] >= 0
    zero = jnp.float32(0.0)
    r = jnp.where(cond, r_ref[...], zero)
    g = jnp.where(cond, g_ref[...], zero)
    b = jnp.where(cond, b_ref[...], zero)
    s = jnp.where(cond, s_ref[...], zero)
    ro_ref[...] = jax.nn.sigmoid(r)
    go_ref[...] = jax.nn.sigmoid(g)
    bo_ref[...] = jax.nn.sigmoid(b)
    so_ref[...] = jnp.maximum(s, zero)


_act_kernel = pl.pallas_call(
    _act_body,
    grid=(_GRID2,),
    in_specs=[
        pl.BlockSpec((_R2, 128), lambda i: (i, 0)),
        pl.BlockSpec((_R2, 128), lambda i: (i, 0)),
        pl.BlockSpec((_R2, 128), lambda i: (i + _GRID2, 0)),
        pl.BlockSpec((_R2, 128), lambda i: (i + 2 * _GRID2, 0)),
        pl.BlockSpec((_R2, 128), lambda i: (i + 3 * _GRID2, 0)),
    ],
    out_specs=[pl.BlockSpec((_R2, 128), lambda i: (i, 0))] * 4,
    out_shape=[jax.ShapeDtypeStruct((MROWS, 128), jnp.float32)] * 4,
)


def kernel(xyz, voxels):
    x = xyz[:, 0].reshape(MROWS, 128)
    y = xyz[:, 1].reshape(MROWS, 128)
    z = xyz[:, 2].reshape(MROWS, 128)
    vox_lin = (voxels.reshape(NB, NB, 2, 128, 4)
               .transpose(0, 1, 2, 4, 3)
               .reshape(NB * NB * NB * 4))
    idx = _idx_kernel(x, y, z)
    g4 = _gather_sc(idx.reshape(N_PTS), vox_lin)
    g4v = g4.reshape(4 * MROWS, 128)
    rp, gp, bp, sp = _act_kernel(idx, g4v, g4v, g4v, g4v)
    rgb = jnp.stack(
        [rp.reshape(N_PTS), gp.reshape(N_PTS), bp.reshape(N_PTS)], axis=1)
    return rgb, sp.reshape(N_PTS, 1)


# speedup vs baseline: 6.7849x; 1.0672x over previous
import functools

import jax
import jax.numpy as jnp
from jax import lax
from jax.experimental import pallas as pl
from jax.experimental.pallas import tpu as pltpu
from jax.experimental.pallas import tpu_sc as plsc

NB = 256
N_PTS = 2097152
MROWS = N_PTS // 128


_R1 = 256
_GRID1 = MROWS // _R1


def _idx_body(x_ref, y_ref, z_ref, idx_ref):
    x = x_ref[...]
    y = y_ref[...]
    z = z_ref[...]
    cond = ((jnp.abs(x) < 0.5) & (jnp.abs(y) < 0.5) & (jnp.abs(z) < 0.5))
    ix = jnp.clip((x * 256.0 + 128.0).astype(jnp.int32), 0, NB - 1)
    iy = jnp.clip((y * 256.0 + 128.0).astype(jnp.int32), 0, NB - 1)
    iz = jnp.clip((z * 256.0 + 128.0).astype(jnp.int32), 0, NB - 1)
    base = (ix * NB + iy) * 1024 + (iz >> 7) * 512 + (iz & 127)
    idx_ref[...] = jnp.where(cond, base, -1)


_idx_kernel = pl.pallas_call(
    _idx_body,
    grid=(_GRID1,),
    in_specs=[pl.BlockSpec((_R1, 128), lambda i: (i, 0))] * 3,
    out_specs=pl.BlockSpec((_R1, 128), lambda i: (i, 0)),
    out_shape=jax.ShapeDtypeStruct((MROWS, 128), jnp.int32),
)


NC, NS, L = 2, 16, 16
NW = NC * NS
PER_W = N_PTS // NW
CHUNK = 4096
N_CHUNKS = PER_W // CHUNK
N_PAIRS = N_CHUNKS // 2
GROUPS = CHUNK // L
VOXN = NB * NB * NB * 4
GATHER_SEG = 512
N_SEG = CHUNK // GATHER_SEG

_mesh = plsc.VectorSubcoreMesh(
    core_axis_name="c", subcore_axis_name="s", num_cores=NC, num_subcores=NS
)


@functools.partial(
    pl.kernel,
    out_type=jax.ShapeDtypeStruct((4, N_PTS), jnp.float32),
    mesh=_mesh,
    scratch_types=[
        pltpu.VMEM((2, CHUNK), jnp.int32),
        pltpu.VMEM((2, CHUNK), jnp.float32),
        pltpu.VMEM((2, CHUNK), jnp.float32),
        pltpu.VMEM((2, CHUNK), jnp.float32),
        pltpu.VMEM((2, CHUNK), jnp.float32),
        pltpu.SemaphoreType.DMA,
        pltpu.SemaphoreType.DMA,
        pltpu.SemaphoreType.DMA,
        pltpu.SemaphoreType.DMA,
        pltpu.SemaphoreType.DMA,
        pltpu.SemaphoreType.DMA,
    ],
    compiler_params=pltpu.CompilerParams(use_tc_tiling_on_sc=False),
)
def _gather_sc(idx_hbm, vox_hbm, out_hbm,
               ib_v, c0_v, c1_v, c2_v, c3_v,
               isem0, isem1, gsem0, gsem1, osem0, osem1):
    wid = lax.axis_index("s") * NC + lax.axis_index("c")
    isem = (isem0, isem1)
    gsem = (gsem0, gsem1)
    osem = (osem0, osem1)

    def cbase(ci):
        return wid * PER_W + ci * CHUNK

    def gather_copies(par, ci):
        out = []
        for c in range(4):
            dst = (c0_v, c1_v, c2_v, c3_v)[c]
            tbl = vox_hbm.at[pl.ds(128 * c, VOXN - 128 * c)]
            for j in range(N_SEG):
                sl = pl.ds(j * GATHER_SEG, GATHER_SEG)
                out.append(pltpu.make_async_copy(
                    tbl.at[plsc.Indices(ib_v.at[par].at[sl],
                                        ignored_value=-1)],
                    dst.at[par].at[sl], gsem[par]))
        return out

    def out_copies(par, ci):
        return [pltpu.make_async_copy(
                    (c0_v, c1_v, c2_v, c3_v)[c].at[par],
                    out_hbm.at[c, pl.ds(cbase(ci), CHUNK)], osem[par])
                for c in range(4)]

    def in_copy(par, ci):
        return pltpu.make_async_copy(
            idx_hbm.at[pl.ds(cbase(ci), CHUNK)], ib_v.at[par], isem[par])

    in_copy(0, 0).start()

    def pair_body(p, carry):
        for par in (0, 1):
            ci = 2 * p + par
            oth = 1 - par
            in_copy(par, ci).wait()
            @pl.when(ci >= 2)
            def _():
                for cp in out_copies(par, ci - 2):
                    cp.wait()
            for cp in gather_copies(par, ci):
                cp.start()
            @pl.when(ci >= 1)
            def _():
                for cp in gather_copies(oth, ci - 1):
                    cp.wait()
                for cp in out_copies(oth, ci - 1):
                    cp.start()
            @pl.when(ci + 1 <= N_CHUNKS - 1)
            def _():
                in_copy(oth, ci + 1).start()
        return carry

    lax.fori_loop(0, N_PAIRS, pair_body, 0)

    last = N_CHUNKS - 1
    for cp in gather_copies(1, last):
        cp.wait()
    for cp in out_copies(1, last):
        cp.start()
    for cp in out_copies(0, last - 1):
        cp.wait()
    for cp in out_copies(1, last):
        cp.wait()


_R2 = 256
_GRID2 = MROWS // _R2


def _act_body(idx_ref, r_ref, g_ref, b_ref, s_ref,
              ro_ref, go_ref, bo_ref, so_ref):
    cond = idx_ref[...] >= 0
    zero = jnp.float32(0.0)
    r = jnp.where(cond, r_ref[...], zero)
    g = jnp.where(cond, g_ref[...], zero)
    b = jnp.where(cond, b_ref[...], zero)
    s = jnp.where(cond, s_ref[...], zero)
    ro_ref[...] = jax.nn.sigmoid(r)
    go_ref[...] = jax.nn.sigmoid(g)
    bo_ref[...] = jax.nn.sigmoid(b)
    so_ref[...] = jnp.maximum(s, zero)


_act_kernel = pl.pallas_call(
    _act_body,
    grid=(_GRID2,),
    in_specs=[
        pl.BlockSpec((_R2, 128), lambda i: (i, 0)),
        pl.BlockSpec((_R2, 128), lambda i: (i, 0)),
        pl.BlockSpec((_R2, 128), lambda i: (i + _GRID2, 0)),
        pl.BlockSpec((_R2, 128), lambda i: (i + 2 * _GRID2, 0)),
        pl.BlockSpec((_R2, 128), lambda i: (i + 3 * _GRID2, 0)),
    ],
    out_specs=[pl.BlockSpec((_R2, 128), lambda i: (i, 0))] * 4,
    out_shape=[jax.ShapeDtypeStruct((MROWS, 128), jnp.float32)] * 4,
)


def kernel(xyz, voxels):
    x = xyz[:, 0].reshape(MROWS, 128)
    y = xyz[:, 1].reshape(MROWS, 128)
    z = xyz[:, 2].reshape(MROWS, 128)
    vox_lin = (voxels.reshape(NB, NB, 2, 128, 4)
               .transpose(0, 1, 2, 4, 3)
               .reshape(NB * NB * NB * 4))
    idx = _idx_kernel(x, y, z)
    g4 = _gather_sc(idx.reshape(N_PTS), vox_lin)
    g4v = g4.reshape(4 * MROWS, 128)
    rp, gp, bp, sp = _act_kernel(idx, g4v, g4v, g4v, g4v)
    rgb = jnp.stack(
        [rp.reshape(N_PTS), gp.reshape(N_PTS), bp.reshape(N_PTS)], axis=1)
    return rgb, sp.reshape(N_PTS, 1)


# speedup vs baseline: 7.3873x; 1.0888x over previous
import functools

import jax
import jax.numpy as jnp
from jax import lax
from jax.experimental import pallas as pl
from jax.experimental.pallas import tpu as pltpu
from jax.experimental.pallas import tpu_sc as plsc

NB = 256
N_PTS = 2097152
MROWS = N_PTS // 128


_R1 = 512
_GRID1 = MROWS // _R1


def _idx_body(x_ref, y_ref, z_ref, idx_ref):
    x = x_ref[...]
    y = y_ref[...]
    z = z_ref[...]
    cond = ((jnp.abs(x) < 0.5) & (jnp.abs(y) < 0.5) & (jnp.abs(z) < 0.5))
    ix = jnp.clip((x * 256.0 + 128.0).astype(jnp.int32), 0, NB - 1)
    iy = jnp.clip((y * 256.0 + 128.0).astype(jnp.int32), 0, NB - 1)
    iz = jnp.clip((z * 256.0 + 128.0).astype(jnp.int32), 0, NB - 1)
    base = (ix * NB + iy) * 1024 + (iz >> 7) * 512 + (iz & 127)
    idx_ref[...] = jnp.where(cond, base, -1)


_idx_kernel = pl.pallas_call(
    _idx_body,
    grid=(_GRID1,),
    in_specs=[pl.BlockSpec((_R1, 128), lambda i: (i, 0))] * 3,
    out_specs=pl.BlockSpec((_R1, 128), lambda i: (i, 0)),
    out_shape=jax.ShapeDtypeStruct((MROWS, 128), jnp.int32),
)


NC, NS, L = 2, 16, 16
NW = NC * NS
PER_W = N_PTS // NW
CHUNK = 4096
N_CHUNKS = PER_W // CHUNK
N_PAIRS = N_CHUNKS // 2
GROUPS = CHUNK // L
VOXN = NB * NB * NB * 4
GATHER_SEG = 512
N_SEG = CHUNK // GATHER_SEG

_mesh = plsc.VectorSubcoreMesh(
    core_axis_name="c", subcore_axis_name="s", num_cores=NC, num_subcores=NS
)


@functools.partial(
    pl.kernel,
    out_type=jax.ShapeDtypeStruct((4, N_PTS), jnp.float32),
    mesh=_mesh,
    scratch_types=[
        pltpu.VMEM((2, CHUNK), jnp.int32),
        pltpu.VMEM((2, CHUNK), jnp.float32),
        pltpu.VMEM((2, CHUNK), jnp.float32),
        pltpu.VMEM((2, CHUNK), jnp.float32),
        pltpu.VMEM((2, CHUNK), jnp.float32),
        pltpu.SemaphoreType.DMA,
        pltpu.SemaphoreType.DMA,
        pltpu.SemaphoreType.DMA,
        pltpu.SemaphoreType.DMA,
        pltpu.SemaphoreType.DMA,
        pltpu.SemaphoreType.DMA,
    ],
    compiler_params=pltpu.CompilerParams(use_tc_tiling_on_sc=False),
)
def _gather_sc(idx_hbm, vox_hbm, out_hbm,
               ib_v, c0_v, c1_v, c2_v, c3_v,
               isem0, isem1, gsem0, gsem1, osem0, osem1):
    wid = lax.axis_index("s") * NC + lax.axis_index("c")
    isem = (isem0, isem1)
    gsem = (gsem0, gsem1)
    osem = (osem0, osem1)

    def cbase(ci):
        return wid * PER_W + ci * CHUNK

    def gather_copies(par, ci):
        out = []
        for c in range(4):
            dst = (c0_v, c1_v, c2_v, c3_v)[c]
            tbl = vox_hbm.at[pl.ds(128 * c, VOXN - 128 * c)]
            for j in range(N_SEG):
                sl = pl.ds(j * GATHER_SEG, GATHER_SEG)
                out.append(pltpu.make_async_copy(
                    tbl.at[plsc.Indices(ib_v.at[par].at[sl],
                                        ignored_value=-1)],
                    dst.at[par].at[sl], gsem[par]))
        return out

    def out_copies(par, ci):
        return [pltpu.make_async_copy(
                    (c0_v, c1_v, c2_v, c3_v)[c].at[par],
                    out_hbm.at[c, pl.ds(cbase(ci), CHUNK)], osem[par])
                for c in range(4)]

    def in_copy(par, ci):
        return pltpu.make_async_copy(
            idx_hbm.at[pl.ds(cbase(ci), CHUNK)], ib_v.at[par], isem[par])

    in_copy(0, 0).start()

    def pair_body(p, carry):
        for par in (0, 1):
            ci = 2 * p + par
            oth = 1 - par
            in_copy(par, ci).wait()
            @pl.when(ci >= 2)
            def _():
                for cp in out_copies(par, ci - 2):
                    cp.wait()
            for cp in gather_copies(par, ci):
                cp.start()
            @pl.when(ci >= 1)
            def _():
                for cp in gather_copies(oth, ci - 1):
                    cp.wait()
                for cp in out_copies(oth, ci - 1):
                    cp.start()
            @pl.when(ci + 1 <= N_CHUNKS - 1)
            def _():
                in_copy(oth, ci + 1).start()
        return carry

    lax.fori_loop(0, N_PAIRS, pair_body, 0)

    last = N_CHUNKS - 1
    for cp in gather_copies(1, last):
        cp.wait()
    for cp in out_copies(1, last):
        cp.start()
    for cp in out_copies(0, last - 1):
        cp.wait()
    for cp in out_copies(1, last):
        cp.wait()


_R2 = 512
_GRID2 = MROWS // _R2


def _act_body(idx_ref, r_ref, g_ref, b_ref, s_ref,
              ro_ref, go_ref, bo_ref, so_ref):
    cond = idx_ref[...] >= 0
    zero = jnp.float32(0.0)
    r = jnp.where(cond, r_ref[...], zero)
    g = jnp.where(cond, g_ref[...], zero)
    b = jnp.where(cond, b_ref[...], zero)
    s = jnp.where(cond, s_ref[...], zero)
    ro_ref[...] = jax.nn.sigmoid(r)
    go_ref[...] = jax.nn.sigmoid(g)
    bo_ref[...] = jax.nn.sigmoid(b)
    so_ref[...] = jnp.maximum(s, zero)


_act_kernel = pl.pallas_call(
    _act_body,
    grid=(_GRID2,),
    in_specs=[
        pl.BlockSpec((_R2, 128), lambda i: (i, 0)),
        pl.BlockSpec((_R2, 128), lambda i: (i, 0)),
        pl.BlockSpec((_R2, 128), lambda i: (i + _GRID2, 0)),
        pl.BlockSpec((_R2, 128), lambda i: (i + 2 * _GRID2, 0)),
        pl.BlockSpec((_R2, 128), lambda i: (i + 3 * _GRID2, 0)),
    ],
    out_specs=[pl.BlockSpec((_R2, 128), lambda i: (i, 0))] * 4,
    out_shape=[jax.ShapeDtypeStruct((MROWS, 128), jnp.float32)] * 4,
)


def kernel(xyz, voxels):
    x = xyz[:, 0].reshape(MROWS, 128)
    y = xyz[:, 1].reshape(MROWS, 128)
    z = xyz[:, 2].reshape(MROWS, 128)
    vox_lin = (voxels.reshape(NB, NB, 2, 128, 4)
               .transpose(0, 1, 2, 4, 3)
               .reshape(NB * NB * NB * 4))
    idx = _idx_kernel(x, y, z)
    g4 = _gather_sc(idx.reshape(N_PTS), vox_lin)
    g4v = g4.reshape(4 * MROWS, 128)
    rp, gp, bp, sp = _act_kernel(idx, g4v, g4v, g4v, g4v)
    rgb = jnp.stack(
        [rp.reshape(N_PTS), gp.reshape(N_PTS), bp.reshape(N_PTS)], axis=1)
    return rgb, sp.reshape(N_PTS, 1)


# speedup vs baseline: 7.7211x; 1.0452x over previous
import functools

import jax
import jax.numpy as jnp
from jax import lax
from jax.experimental import pallas as pl
from jax.experimental.pallas import tpu as pltpu
from jax.experimental.pallas import tpu_sc as plsc

NB = 256
N_PTS = 2097152
MROWS = N_PTS // 128


_R1 = 1024
_GRID1 = MROWS // _R1


def _idx_body(x_ref, y_ref, z_ref, idx_ref):
    x = x_ref[...]
    y = y_ref[...]
    z = z_ref[...]
    cond = ((jnp.abs(x) < 0.5) & (jnp.abs(y) < 0.5) & (jnp.abs(z) < 0.5))
    ix = jnp.clip((x * 256.0 + 128.0).astype(jnp.int32), 0, NB - 1)
    iy = jnp.clip((y * 256.0 + 128.0).astype(jnp.int32), 0, NB - 1)
    iz = jnp.clip((z * 256.0 + 128.0).astype(jnp.int32), 0, NB - 1)
    base = (ix * NB + iy) * 1024 + (iz >> 7) * 512 + (iz & 127)
    idx_ref[...] = jnp.where(cond, base, -1)


_idx_kernel = pl.pallas_call(
    _idx_body,
    grid=(_GRID1,),
    in_specs=[pl.BlockSpec((_R1, 128), lambda i: (i, 0))] * 3,
    out_specs=pl.BlockSpec((_R1, 128), lambda i: (i, 0)),
    out_shape=jax.ShapeDtypeStruct((MROWS, 128), jnp.int32),
)


NC, NS, L = 2, 16, 16
NW = NC * NS
PER_W = N_PTS // NW
CHUNK = 4096
N_CHUNKS = PER_W // CHUNK
N_PAIRS = N_CHUNKS // 2
GROUPS = CHUNK // L
VOXN = NB * NB * NB * 4
GATHER_SEG = 512
N_SEG = CHUNK // GATHER_SEG

_mesh = plsc.VectorSubcoreMesh(
    core_axis_name="c", subcore_axis_name="s", num_cores=NC, num_subcores=NS
)


@functools.partial(
    pl.kernel,
    out_type=jax.ShapeDtypeStruct((4, N_PTS), jnp.float32),
    mesh=_mesh,
    scratch_types=[
        pltpu.VMEM((2, CHUNK), jnp.int32),
        pltpu.VMEM((2, CHUNK), jnp.float32),
        pltpu.VMEM((2, CHUNK), jnp.float32),
        pltpu.VMEM((2, CHUNK), jnp.float32),
        pltpu.VMEM((2, CHUNK), jnp.float32),
        pltpu.SemaphoreType.DMA,
        pltpu.SemaphoreType.DMA,
        pltpu.SemaphoreType.DMA,
        pltpu.SemaphoreType.DMA,
        pltpu.SemaphoreType.DMA,
        pltpu.SemaphoreType.DMA,
    ],
    compiler_params=pltpu.CompilerParams(use_tc_tiling_on_sc=False),
)
def _gather_sc(idx_hbm, vox_hbm, out_hbm,
               ib_v, c0_v, c1_v, c2_v, c3_v,
               isem0, isem1, gsem0, gsem1, osem0, osem1):
    wid = lax.axis_index("s") * NC + lax.axis_index("c")
    isem = (isem0, isem1)
    gsem = (gsem0, gsem1)
    osem = (osem0, osem1)

    def cbase(ci):
        return wid * PER_W + ci * CHUNK

    def gather_copies(par, ci):
        out = []
        for c in range(4):
            dst = (c0_v, c1_v, c2_v, c3_v)[c]
            tbl = vox_hbm.at[pl.ds(128 * c, VOXN - 128 * c)]
            for j in range(N_SEG):
                sl = pl.ds(j * GATHER_SEG, GATHER_SEG)
                out.append(pltpu.make_async_copy(
                    tbl.at[plsc.Indices(ib_v.at[par].at[sl],
                                        ignored_value=-1)],
                    dst.at[par].at[sl], gsem[par]))
        return out

    def out_copies(par, ci):
        return [pltpu.make_async_copy(
                    (c0_v, c1_v, c2_v, c3_v)[c].at[par],
                    out_hbm.at[c, pl.ds(cbase(ci), CHUNK)], osem[par])
                for c in range(4)]

    def in_copy(par, ci):
        return pltpu.make_async_copy(
            idx_hbm.at[pl.ds(cbase(ci), CHUNK)], ib_v.at[par], isem[par])

    in_copy(0, 0).start()

    def pair_body(p, carry):
        for par in (0, 1):
            ci = 2 * p + par
            oth = 1 - par
            in_copy(par, ci).wait()
            @pl.when(ci >= 2)
            def _():
                for cp in out_copies(par, ci - 2):
                    cp.wait()
            for cp in gather_copies(par, ci):
                cp.start()
            @pl.when(ci >= 1)
            def _():
                for cp in gather_copies(oth, ci - 1):
                    cp.wait()
                for cp in out_copies(oth, ci - 1):
                    cp.start()
            @pl.when(ci + 1 <= N_CHUNKS - 1)
            def _():
                in_copy(oth, ci + 1).start()
        return carry

    lax.fori_loop(0, N_PAIRS, pair_body, 0)

    last = N_CHUNKS - 1
    for cp in gather_copies(1, last):
        cp.wait()
    for cp in out_copies(1, last):
        cp.start()
    for cp in out_copies(0, last - 1):
        cp.wait()
    for cp in out_copies(1, last):
        cp.wait()


_R2 = 1024
_GRID2 = MROWS // _R2


def _act_body(idx_ref, r_ref, g_ref, b_ref, s_ref,
              ro_ref, go_ref, bo_ref, so_ref):
    cond = idx_ref[...] >= 0
    zero = jnp.float32(0.0)
    r = jnp.where(cond, r_ref[...], zero)
    g = jnp.where(cond, g_ref[...], zero)
    b = jnp.where(cond, b_ref[...], zero)
    s = jnp.where(cond, s_ref[...], zero)
    ro_ref[...] = jax.nn.sigmoid(r)
    go_ref[...] = jax.nn.sigmoid(g)
    bo_ref[...] = jax.nn.sigmoid(b)
    so_ref[...] = jnp.maximum(s, zero)


_act_kernel = pl.pallas_call(
    _act_body,
    grid=(_GRID2,),
    in_specs=[
        pl.BlockSpec((_R2, 128), lambda i: (i, 0)),
        pl.BlockSpec((_R2, 128), lambda i: (i, 0)),
        pl.BlockSpec((_R2, 128), lambda i: (i + _GRID2, 0)),
        pl.BlockSpec((_R2, 128), lambda i: (i + 2 * _GRID2, 0)),
        pl.BlockSpec((_R2, 128), lambda i: (i + 3 * _GRID2, 0)),
    ],
    out_specs=[pl.BlockSpec((_R2, 128), lambda i: (i, 0))] * 4,
    out_shape=[jax.ShapeDtypeStruct((MROWS, 128), jnp.float32)] * 4,
)


def kernel(xyz, voxels):
    x = xyz[:, 0].reshape(MROWS, 128)
    y = xyz[:, 1].reshape(MROWS, 128)
    z = xyz[:, 2].reshape(MROWS, 128)
    vox_lin = (voxels.reshape(NB, NB, 2, 128, 4)
               .transpose(0, 1, 2, 4, 3)
               .reshape(NB * NB * NB * 4))
    idx = _idx_kernel(x, y, z)
    g4 = _gather_sc(idx.reshape(N_PTS), vox_lin)
    g4v = g4.reshape(4 * MROWS, 128)
    rp, gp, bp, sp = _act_kernel(idx, g4v, g4v, g4v, g4v)
    rgb = jnp.stack(
        [rp.reshape(N_PTS), gp.reshape(N_PTS), bp.reshape(N_PTS)], axis=1)
    return rgb, sp.reshape(N_PTS, 1)


# speedup vs baseline: 7.8913x; 1.0220x over previous
import functools

import jax
import jax.numpy as jnp
from jax import lax
from jax.experimental import pallas as pl
from jax.experimental.pallas import tpu as pltpu
from jax.experimental.pallas import tpu_sc as plsc

NB = 256
N_PTS = 2097152
MROWS = N_PTS // 128


_R1 = 2048
_GRID1 = MROWS // _R1


def _idx_body(x_ref, y_ref, z_ref, idx_ref):
    x = x_ref[...]
    y = y_ref[...]
    z = z_ref[...]
    cond = ((jnp.abs(x) < 0.5) & (jnp.abs(y) < 0.5) & (jnp.abs(z) < 0.5))
    ix = jnp.clip((x * 256.0 + 128.0).astype(jnp.int32), 0, NB - 1)
    iy = jnp.clip((y * 256.0 + 128.0).astype(jnp.int32), 0, NB - 1)
    iz = jnp.clip((z * 256.0 + 128.0).astype(jnp.int32), 0, NB - 1)
    base = (ix * NB + iy) * 1024 + (iz >> 7) * 512 + (iz & 127)
    idx_ref[...] = jnp.where(cond, base, -1)


_idx_kernel = pl.pallas_call(
    _idx_body,
    grid=(_GRID1,),
    in_specs=[pl.BlockSpec((_R1, 128), lambda i: (i, 0))] * 3,
    out_specs=pl.BlockSpec((_R1, 128), lambda i: (i, 0)),
    out_shape=jax.ShapeDtypeStruct((MROWS, 128), jnp.int32),
)


NC, NS, L = 2, 16, 16
NW = NC * NS
PER_W = N_PTS // NW
CHUNK = 4096
N_CHUNKS = PER_W // CHUNK
N_PAIRS = N_CHUNKS // 2
GROUPS = CHUNK // L
VOXN = NB * NB * NB * 4
GATHER_SEG = 512
N_SEG = CHUNK // GATHER_SEG

_mesh = plsc.VectorSubcoreMesh(
    core_axis_name="c", subcore_axis_name="s", num_cores=NC, num_subcores=NS
)


@functools.partial(
    pl.kernel,
    out_type=jax.ShapeDtypeStruct((4, N_PTS), jnp.float32),
    mesh=_mesh,
    scratch_types=[
        pltpu.VMEM((2, CHUNK), jnp.int32),
        pltpu.VMEM((2, CHUNK), jnp.float32),
        pltpu.VMEM((2, CHUNK), jnp.float32),
        pltpu.VMEM((2, CHUNK), jnp.float32),
        pltpu.VMEM((2, CHUNK), jnp.float32),
        pltpu.SemaphoreType.DMA,
        pltpu.SemaphoreType.DMA,
        pltpu.SemaphoreType.DMA,
        pltpu.SemaphoreType.DMA,
        pltpu.SemaphoreType.DMA,
        pltpu.SemaphoreType.DMA,
    ],
    compiler_params=pltpu.CompilerParams(use_tc_tiling_on_sc=False),
)
def _gather_sc(idx_hbm, vox_hbm, out_hbm,
               ib_v, c0_v, c1_v, c2_v, c3_v,
               isem0, isem1, gsem0, gsem1, osem0, osem1):
    wid = lax.axis_index("s") * NC + lax.axis_index("c")
    isem = (isem0, isem1)
    gsem = (gsem0, gsem1)
    osem = (osem0, osem1)

    def cbase(ci):
        return wid * PER_W + ci * CHUNK

    def gather_copies(par, ci):
        out = []
        for c in range(4):
            dst = (c0_v, c1_v, c2_v, c3_v)[c]
            tbl = vox_hbm.at[pl.ds(128 * c, VOXN - 128 * c)]
            for j in range(N_SEG):
                sl = pl.ds(j * GATHER_SEG, GATHER_SEG)
                out.append(pltpu.make_async_copy(
                    tbl.at[plsc.Indices(ib_v.at[par].at[sl],
                                        ignored_value=-1)],
                    dst.at[par].at[sl], gsem[par]))
        return out

    def out_copies(par, ci):
        return [pltpu.make_async_copy(
                    (c0_v, c1_v, c2_v, c3_v)[c].at[par],
                    out_hbm.at[c, pl.ds(cbase(ci), CHUNK)], osem[par])
                for c in range(4)]

    def in_copy(par, ci):
        return pltpu.make_async_copy(
            idx_hbm.at[pl.ds(cbase(ci), CHUNK)], ib_v.at[par], isem[par])

    in_copy(0, 0).start()

    def pair_body(p, carry):
        for par in (0, 1):
            ci = 2 * p + par
            oth = 1 - par
            in_copy(par, ci).wait()
            @pl.when(ci >= 2)
            def _():
                for cp in out_copies(par, ci - 2):
                    cp.wait()
            for cp in gather_copies(par, ci):
                cp.start()
            @pl.when(ci >= 1)
            def _():
                for cp in gather_copies(oth, ci - 1):
                    cp.wait()
                for cp in out_copies(oth, ci - 1):
                    cp.start()
            @pl.when(ci + 1 <= N_CHUNKS - 1)
            def _():
                in_copy(oth, ci + 1).start()
        return carry

    lax.fori_loop(0, N_PAIRS, pair_body, 0)

    last = N_CHUNKS - 1
    for cp in gather_copies(1, last):
        cp.wait()
    for cp in out_copies(1, last):
        cp.start()
    for cp in out_copies(0, last - 1):
        cp.wait()
    for cp in out_copies(1, last):
        cp.wait()


_R2 = 2048
_GRID2 = MROWS // _R2


def _act_body(idx_ref, r_ref, g_ref, b_ref, s_ref,
              ro_ref, go_ref, bo_ref, so_ref):
    cond = idx_ref[...] >= 0
    zero = jnp.float32(0.0)
    r = jnp.where(cond, r_ref[...], zero)
    g = jnp.where(cond, g_ref[...], zero)
    b = jnp.where(cond, b_ref[...], zero)
    s = jnp.where(cond, s_ref[...], zero)
    ro_ref[...] = jax.nn.sigmoid(r)
    go_ref[...] = jax.nn.sigmoid(g)
    bo_ref[...] = jax.nn.sigmoid(b)
    so_ref[...] = jnp.maximum(s, zero)


_act_kernel = pl.pallas_call(
    _act_body,
    grid=(_GRID2,),
    in_specs=[
        pl.BlockSpec((_R2, 128), lambda i: (i, 0)),
        pl.BlockSpec((_R2, 128), lambda i: (i, 0)),
        pl.BlockSpec((_R2, 128), lambda i: (i + _GRID2, 0)),
        pl.BlockSpec((_R2, 128), lambda i: (i + 2 * _GRID2, 0)),
        pl.BlockSpec((_R2, 128), lambda i: (i + 3 * _GRID2, 0)),
    ],
    out_specs=[pl.BlockSpec((_R2, 128), lambda i: (i, 0))] * 4,
    out_shape=[jax.ShapeDtypeStruct((MROWS, 128), jnp.float32)] * 4,
)


def kernel(xyz, voxels):
    x = xyz[:, 0].reshape(MROWS, 128)
    y = xyz[:, 1].reshape(MROWS, 128)
    z = xyz[:, 2].reshape(MROWS, 128)
    vox_lin = (voxels.reshape(NB, NB, 2, 128, 4)
               .transpose(0, 1, 2, 4, 3)
               .reshape(NB * NB * NB * 4))
    idx = _idx_kernel(x, y, z)
    g4 = _gather_sc(idx.reshape(N_PTS), vox_lin)
    g4v = g4.reshape(4 * MROWS, 128)
    rp, gp, bp, sp = _act_kernel(idx, g4v, g4v, g4v, g4v)
    rgb = jnp.stack(
        [rp.reshape(N_PTS), gp.reshape(N_PTS), bp.reshape(N_PTS)], axis=1)
    return rgb, sp.reshape(N_PTS, 1)


# speedup vs baseline: 7.9564x; 1.0082x over previous
import functools

import jax
import jax.numpy as jnp
from jax import lax
from jax.experimental import pallas as pl
from jax.experimental.pallas import tpu as pltpu
from jax.experimental.pallas import tpu_sc as plsc

NB = 256
N_PTS = 2097152
MROWS = N_PTS // 128


_R1 = 4096
_GRID1 = MROWS // _R1


def _idx_body(x_ref, y_ref, z_ref, idx_ref):
    x = x_ref[...]
    y = y_ref[...]
    z = z_ref[...]
    cond = ((jnp.abs(x) < 0.5) & (jnp.abs(y) < 0.5) & (jnp.abs(z) < 0.5))
    ix = jnp.clip((x * 256.0 + 128.0).astype(jnp.int32), 0, NB - 1)
    iy = jnp.clip((y * 256.0 + 128.0).astype(jnp.int32), 0, NB - 1)
    iz = jnp.clip((z * 256.0 + 128.0).astype(jnp.int32), 0, NB - 1)
    base = (ix * NB + iy) * 1024 + (iz >> 7) * 512 + (iz & 127)
    idx_ref[...] = jnp.where(cond, base, -1)


_idx_kernel = pl.pallas_call(
    _idx_body,
    grid=(_GRID1,),
    in_specs=[pl.BlockSpec((_R1, 128), lambda i: (i, 0))] * 3,
    out_specs=pl.BlockSpec((_R1, 128), lambda i: (i, 0)),
    out_shape=jax.ShapeDtypeStruct((MROWS, 128), jnp.int32),
)


NC, NS, L = 2, 16, 16
NW = NC * NS
PER_W = N_PTS // NW
CHUNK = 4096
N_CHUNKS = PER_W // CHUNK
N_PAIRS = N_CHUNKS // 2
GROUPS = CHUNK // L
VOXN = NB * NB * NB * 4
GATHER_SEG = 512
N_SEG = CHUNK // GATHER_SEG

_mesh = plsc.VectorSubcoreMesh(
    core_axis_name="c", subcore_axis_name="s", num_cores=NC, num_subcores=NS
)


@functools.partial(
    pl.kernel,
    out_type=jax.ShapeDtypeStruct((4, N_PTS), jnp.float32),
    mesh=_mesh,
    scratch_types=[
        pltpu.VMEM((2, CHUNK), jnp.int32),
        pltpu.VMEM((2, CHUNK), jnp.float32),
        pltpu.VMEM((2, CHUNK), jnp.float32),
        pltpu.VMEM((2, CHUNK), jnp.float32),
        pltpu.VMEM((2, CHUNK), jnp.float32),
        pltpu.SemaphoreType.DMA,
        pltpu.SemaphoreType.DMA,
        pltpu.SemaphoreType.DMA,
        pltpu.SemaphoreType.DMA,
        pltpu.SemaphoreType.DMA,
        pltpu.SemaphoreType.DMA,
    ],
    compiler_params=pltpu.CompilerParams(use_tc_tiling_on_sc=False),
)
def _gather_sc(idx_hbm, vox_hbm, out_hbm,
               ib_v, c0_v, c1_v, c2_v, c3_v,
               isem0, isem1, gsem0, gsem1, osem0, osem1):
    wid = lax.axis_index("s") * NC + lax.axis_index("c")
    isem = (isem0, isem1)
    gsem = (gsem0, gsem1)
    osem = (osem0, osem1)

    def cbase(ci):
        return wid * PER_W + ci * CHUNK

    def gather_copies(par, ci):
        out = []
        for c in range(4):
            dst = (c0_v, c1_v, c2_v, c3_v)[c]
            tbl = vox_hbm.at[pl.ds(128 * c, VOXN - 128 * c)]
            for j in range(N_SEG):
                sl = pl.ds(j * GATHER_SEG, GATHER_SEG)
                out.append(pltpu.make_async_copy(
                    tbl.at[plsc.Indices(ib_v.at[par].at[sl],
                                        ignored_value=-1)],
                    dst.at[par].at[sl], gsem[par]))
        return out

    def out_copies(par, ci):
        return [pltpu.make_async_copy(
                    (c0_v, c1_v, c2_v, c3_v)[c].at[par],
                    out_hbm.at[c, pl.ds(cbase(ci), CHUNK)], osem[par])
                for c in range(4)]

    def in_copy(par, ci):
        return pltpu.make_async_copy(
            idx_hbm.at[pl.ds(cbase(ci), CHUNK)], ib_v.at[par], isem[par])

    in_copy(0, 0).start()

    def pair_body(p, carry):
        for par in (0, 1):
            ci = 2 * p + par
            oth = 1 - par
            in_copy(par, ci).wait()
            @pl.when(ci >= 2)
            def _():
                for cp in out_copies(par, ci - 2):
                    cp.wait()
            for cp in gather_copies(par, ci):
                cp.start()
            @pl.when(ci >= 1)
            def _():
                for cp in gather_copies(oth, ci - 1):
                    cp.wait()
                for cp in out_copies(oth, ci - 1):
                    cp.start()
            @pl.when(ci + 1 <= N_CHUNKS - 1)
            def _():
                in_copy(oth, ci + 1).start()
        return carry

    lax.fori_loop(0, N_PAIRS, pair_body, 0)

    last = N_CHUNKS - 1
    for cp in gather_copies(1, last):
        cp.wait()
    for cp in out_copies(1, last):
        cp.start()
    for cp in out_copies(0, last - 1):
        cp.wait()
    for cp in out_copies(1, last):
        cp.wait()


_R2 = 4096
_GRID2 = MROWS // _R2


def _act_body(idx_ref, r_ref, g_ref, b_ref, s_ref,
              ro_ref, go_ref, bo_ref, so_ref):
    cond = idx_ref[...] >= 0
    zero = jnp.float32(0.0)
    r = jnp.where(cond, r_ref[...], zero)
    g = jnp.where(cond, g_ref[...], zero)
    b = jnp.where(cond, b_ref[...], zero)
    s = jnp.where(cond, s_ref[...], zero)
    ro_ref[...] = jax.nn.sigmoid(r)
    go_ref[...] = jax.nn.sigmoid(g)
    bo_ref[...] = jax.nn.sigmoid(b)
    so_ref[...] = jnp.maximum(s, zero)


_act_kernel = pl.pallas_call(
    _act_body,
    grid=(_GRID2,),
    in_specs=[
        pl.BlockSpec((_R2, 128), lambda i: (i, 0)),
        pl.BlockSpec((_R2, 128), lambda i: (i, 0)),
        pl.BlockSpec((_R2, 128), lambda i: (i + _GRID2, 0)),
        pl.BlockSpec((_R2, 128), lambda i: (i + 2 * _GRID2, 0)),
        pl.BlockSpec((_R2, 128), lambda i: (i + 3 * _GRID2, 0)),
    ],
    out_specs=[pl.BlockSpec((_R2, 128), lambda i: (i, 0))] * 4,
    out_shape=[jax.ShapeDtypeStruct((MROWS, 128), jnp.float32)] * 4,
)


def kernel(xyz, voxels):
    x = xyz[:, 0].reshape(MROWS, 128)
    y = xyz[:, 1].reshape(MROWS, 128)
    z = xyz[:, 2].reshape(MROWS, 128)
    vox_lin = (voxels.reshape(NB, NB, 2, 128, 4)
               .transpose(0, 1, 2, 4, 3)
               .reshape(NB * NB * NB * 4))
    idx = _idx_kernel(x, y, z)
    g4 = _gather_sc(idx.reshape(N_PTS), vox_lin)
    g4v = g4.reshape(4 * MROWS, 128)
    rp, gp, bp, sp = _act_kernel(idx, g4v, g4v, g4v, g4v)
    rgb = jnp.stack(
        [rp.reshape(N_PTS), gp.reshape(N_PTS), bp.reshape(N_PTS)], axis=1)
    return rgb, sp.reshape(N_PTS, 1)
